# baseline (speedup 1.0000x reference)
import functools

import jax
import jax.numpy as jnp
from jax import lax
from jax.experimental import pallas as pl
from jax.experimental.pallas import tpu as pltpu
from jax.experimental.pallas import tpu_sc as plsc

NUM_CORES = 2
NUM_SUBCORES = 16
NUM_WORKERS = NUM_CORES * NUM_SUBCORES
KBLK = 32
NBUF = 4
LA = 3
W = 1


def _sc_gather(ids, table):
    s, t = ids.shape
    vp, d = table.shape
    upw = s // NUM_WORKERS
    n_blocks = upw // KBLK
    n_bodies = upw // NBUF
    rows_per_tile = vp // NUM_SUBCORES
    t_lo = min(t, 128)
    t_hi = t - t_lo

    mesh = plsc.VectorSubcoreMesh(core_axis_name="c", subcore_axis_name="s")

    @functools.partial(
        pl.kernel,
        mesh=mesh,
        out_type=jax.ShapeDtypeStruct((s, t, d), jnp.float32),
        scratch_types=[
            pltpu.VMEM((2, KBLK, t), jnp.int32),
            pltpu.VMEM((NBUF, t, d), jnp.float32),
            pltpu.VMEM_SHARED((vp, d), jnp.float32),
        ]
        + [pltpu.SemaphoreType.DMA] * (2 * NBUF + 1),
    )
    def k(ids_hbm, table_hbm, out_hbm, idx_v, rows_v, table_sp, *sems):
        gsem = sems[:NBUF]
        osem = sems[NBUF:2 * NBUF]
        isem = sems[2 * NBUF]
        wid = lax.axis_index("s") * NUM_CORES + lax.axis_index("c")
        ubase = wid * upw

        sid = lax.axis_index("s")
        stage = rows_v.at[0, pl.ds(0, rows_per_tile)]
        pltpu.sync_copy(table_hbm.at[pl.ds(sid * rows_per_tile,
                                           rows_per_tile)], stage)
        pltpu.sync_copy(stage, table_sp.at[pl.ds(sid * rows_per_tile,
                                                 rows_per_tile)])
        plsc.subcore_barrier()

        def prefetch_idx(m):
            p = lax.rem(m, 2)
            pltpu.async_copy(ids_hbm.at[pl.ds(ubase + m * KBLK, KBLK)],
                             idx_v.at[p], isem)

        def wait_idx():
            pltpu.make_async_copy(ids_hbm.at[pl.ds(0, KBLK)], idx_v.at[0],
                                  isem).wait()

        def issue_gather(v, b):
            blk = lax.div(v, KBLK)
            p = lax.rem(blk, 2)
            r = lax.rem(v, KBLK)
            pltpu.async_copy(table_sp.at[idx_v.at[p, r, pl.ds(0, t_lo)]],
                             rows_v.at[b, pl.ds(0, t_lo)], gsem[b])
            if t_hi:
                pltpu.async_copy(
                    table_sp.at[idx_v.at[p, r, pl.ds(t_lo, t_hi)]],
                    rows_v.at[b, pl.ds(t_lo, t_hi)], gsem[b])

        def wait_gathers(b):
            pltpu.make_async_copy(table_sp.at[idx_v.at[0, 0]],
                                  rows_v.at[b], gsem[b]).wait()

        def wait_write(b):
            pltpu.make_async_copy(rows_v.at[b], out_hbm.at[0],
                                  osem[b]).wait()

        prefetch_idx(0)
        wait_idx()
        for b in range(LA):
            issue_gather(b, b)

        def body(bt, carry):
            nbpb = n_bodies // n_blocks
            @pl.when(jnp.logical_and(lax.rem(bt, nbpb) == nbpb - 2,
                                     bt != n_bodies - 2))
            def _():
                prefetch_idx(lax.div(NBUF * bt + 2 * NBUF, KBLK))

            @pl.when(jnp.logical_and(lax.rem(bt, nbpb) == nbpb - 1,
                                     bt != n_bodies - 1))
            def _():
                wait_idx()

            for kslot in range(NBUF):
                u = NBUF * bt + kslot
                bn = (kslot + LA) % NBUF
                wait_gathers(kslot)
                pltpu.async_copy(rows_v.at[kslot], out_hbm.at[ubase + u],
                                 osem[kslot])
                @pl.when(u >= W)
                def _():
                    wait_write(bn)

                @pl.when(u + LA < upw)
                def _():
                    issue_gather(u + LA, bn)

            return carry

        lax.fori_loop(0, n_bodies, body, 0)

        for v in range(upw - W, upw):
            wait_write(v % NBUF)

    return k(ids, table)


def kernel(tool_ids, tool_embed_weight):
    v, d = tool_embed_weight.shape
    align = 8 * NUM_SUBCORES
    vp = (v + align - 1) // align * align
    table_p = jnp.pad(tool_embed_weight, ((0, vp - v), (0, 0)))
    return _sc_gather(tool_ids.astype(jnp.int32), table_p)


# speedup vs baseline: 1.0023x; 1.0023x over previous
import functools

import jax
import jax.numpy as jnp
from jax import lax
from jax.experimental import pallas as pl
from jax.experimental.pallas import tpu as pltpu
from jax.experimental.pallas import tpu_sc as plsc

NUM_CORES = 2
NUM_SUBCORES = 16
NUM_WORKERS = NUM_CORES * NUM_SUBCORES
KBLK = 32
NBUF = 4
LA = 2
W = 2


def _sc_gather(ids, table):
    s, t = ids.shape
    vp, d = table.shape
    upw = s // NUM_WORKERS
    n_blocks = upw // KBLK
    n_bodies = upw // NBUF
    rows_per_tile = vp // NUM_SUBCORES
    t_lo = min(t, 128)
    t_hi = t - t_lo

    mesh = plsc.VectorSubcoreMesh(core_axis_name="c", subcore_axis_name="s")

    @functools.partial(
        pl.kernel,
        mesh=mesh,
        out_type=jax.ShapeDtypeStruct((s, t, d), jnp.float32),
        scratch_types=[
            pltpu.VMEM((2, KBLK, t), jnp.int32),
            pltpu.VMEM((NBUF, t, d), jnp.float32),
            pltpu.VMEM_SHARED((vp, d), jnp.float32),
        ]
        + [pltpu.SemaphoreType.DMA] * (2 * NBUF + 1),
    )
    def k(ids_hbm, table_hbm, out_hbm, idx_v, rows_v, table_sp, *sems):
        gsem = sems[:NBUF]
        osem = sems[NBUF:2 * NBUF]
        isem = sems[2 * NBUF]
        wid = lax.axis_index("s") * NUM_CORES + lax.axis_index("c")
        ubase = wid * upw

        sid = lax.axis_index("s")
        stage = rows_v.at[0, pl.ds(0, rows_per_tile)]
        pltpu.sync_copy(table_hbm.at[pl.ds(sid * rows_per_tile,
                                           rows_per_tile)], stage)
        pltpu.sync_copy(stage, table_sp.at[pl.ds(sid * rows_per_tile,
                                                 rows_per_tile)])
        plsc.subcore_barrier()

        def prefetch_idx(m):
            p = lax.rem(m, 2)
            pltpu.async_copy(ids_hbm.at[pl.ds(ubase + m * KBLK, KBLK)],
                             idx_v.at[p], isem)

        def wait_idx():
            pltpu.make_async_copy(ids_hbm.at[pl.ds(0, KBLK)], idx_v.at[0],
                                  isem).wait()

        def issue_gather(v, b):
            blk = lax.div(v, KBLK)
            p = lax.rem(blk, 2)
            r = lax.rem(v, KBLK)
            pltpu.async_copy(table_sp.at[idx_v.at[p, r, pl.ds(0, t_lo)]],
                             rows_v.at[b, pl.ds(0, t_lo)], gsem[b])
            if t_hi:
                pltpu.async_copy(
                    table_sp.at[idx_v.at[p, r, pl.ds(t_lo, t_hi)]],
                    rows_v.at[b, pl.ds(t_lo, t_hi)], gsem[b])

        def wait_gathers(b):
            pltpu.make_async_copy(table_sp.at[idx_v.at[0, 0]],
                                  rows_v.at[b], gsem[b]).wait()

        def wait_write(b):
            pltpu.make_async_copy(rows_v.at[b], out_hbm.at[0],
                                  osem[b]).wait()

        prefetch_idx(0)
        wait_idx()
        for b in range(LA):
            issue_gather(b, b)

        def body(bt, carry):
            nbpb = n_bodies // n_blocks
            @pl.when(jnp.logical_and(lax.rem(bt, nbpb) == nbpb - 2,
                                     bt != n_bodies - 2))
            def _():
                prefetch_idx(lax.div(NBUF * bt + 2 * NBUF, KBLK))

            @pl.when(jnp.logical_and(lax.rem(bt, nbpb) == nbpb - 1,
                                     bt != n_bodies - 1))
            def _():
                wait_idx()

            for kslot in range(NBUF):
                u = NBUF * bt + kslot
                bn = (kslot + LA) % NBUF
                wait_gathers(kslot)
                pltpu.async_copy(rows_v.at[kslot], out_hbm.at[ubase + u],
                                 osem[kslot])
                @pl.when(u >= W)
                def _():
                    wait_write(bn)

                @pl.when(u + LA < upw)
                def _():
                    issue_gather(u + LA, bn)

            return carry

        lax.fori_loop(0, n_bodies, body, 0)

        for v in range(upw - W, upw):
            wait_write(v % NBUF)

    return k(ids, table)


def kernel(tool_ids, tool_embed_weight):
    v, d = tool_embed_weight.shape
    align = 8 * NUM_SUBCORES
    vp = (v + align - 1) // align * align
    table_p = jnp.pad(tool_embed_weight, ((0, vp - v), (0, 0)))
    return _sc_gather(tool_ids.astype(jnp.int32), table_p)
